# baseline (speedup 1.0000x reference)
import functools

import jax
import jax.numpy as jnp
from jax import lax
from jax.experimental import pallas as pl
from jax.experimental.pallas import tpu as pltpu
from jax.experimental.pallas import tpu_sc as plsc

NUM_ITEMS = 100000
REQ_VEC = 400
LATENT = 64
BATCH = 1024

ITEM_BLOCK = 2048


def _make_sc_gather(num_rows, d, b):
    info = plsc.get_sparse_core_info()
    nw = info.num_cores * info.num_subcores
    assert d % info.num_lanes == 0 and b % (8 * nw) == 0
    b_per_w = b // nw
    mesh = plsc.VectorSubcoreMesh(core_axis_name="c", subcore_axis_name="s")

    @functools.partial(
        pl.kernel,
        mesh=mesh,
        out_type=jax.ShapeDtypeStruct((b, d), jnp.float32),
        scratch_types=[
            pltpu.VMEM((b_per_w,), jnp.int32),
            pltpu.VMEM((b_per_w, d), jnp.float32),
            pltpu.SemaphoreType.DMA,
        ],
        compiler_params=pltpu.CompilerParams(use_tc_tiling_on_sc=False),
    )
    def gather_kernel(table_hbm, idx_hbm, out_hbm, idx_v, rows_v, sem):
        wid = lax.axis_index("s") * info.num_cores + lax.axis_index("c")
        base = wid * b_per_w
        pltpu.sync_copy(idx_hbm.at[pl.ds(base, b_per_w)], idx_v)
        pltpu.async_copy(table_hbm.at[idx_v], rows_v, sem).wait()
        pltpu.sync_copy(rows_v, out_hbm.at[pl.ds(base, b_per_w)])

    return gather_kernel


def _score_body(ug_ref, item_ref, fs_ref, out_ref, fu_ref):
    @pl.when(pl.program_id(0) == 0)
    def _():
        fu_ref[...] = jnp.dot(
            ug_ref[...], fs_ref[...], preferred_element_type=jnp.float32
        )

    fi = jnp.dot(item_ref[...], fs_ref[...], preferred_element_type=jnp.float32)
    logits = lax.dot_general(
        fu_ref[...], fi, (((1,), (1,)), ((), ())),
        preferred_element_type=jnp.float32,
    )
    out_ref[...] = jax.nn.sigmoid(logits)


def _tc_score(ug, item_vector, FS, interpret=False):
    n_items = item_vector.shape[0]
    grid = (pl.cdiv(n_items, ITEM_BLOCK),)
    return pl.pallas_call(
        _score_body,
        grid=grid,
        in_specs=[
            pl.BlockSpec((BATCH, REQ_VEC), lambda i: (0, 0)),
            pl.BlockSpec((ITEM_BLOCK, REQ_VEC), lambda i: (i, 0)),
            pl.BlockSpec((REQ_VEC, LATENT), lambda i: (0, 0)),
        ],
        out_specs=pl.BlockSpec((BATCH, ITEM_BLOCK), lambda i: (0, i)),
        out_shape=jax.ShapeDtypeStruct((BATCH, n_items), jnp.float32),
        scratch_shapes=[pltpu.VMEM((BATCH, LATENT), jnp.float32)],
        interpret=interpret,
    )(ug, item_vector, FS)


@jax.jit
def kernel(users, user_vector, item_vector, FS):
    gather = _make_sc_gather(user_vector.shape[0], REQ_VEC, BATCH)
    ug = gather(user_vector, users.astype(jnp.int32))
    return _tc_score(ug, item_vector, FS)


# speedup vs baseline: 1.9994x; 1.9994x over previous
import functools

import jax
import jax.numpy as jnp
from jax import lax
from jax.experimental import pallas as pl
from jax.experimental.pallas import tpu as pltpu
from jax.experimental.pallas import tpu_sc as plsc

REQ_VEC = 400
LATENT = 64
BATCH = 1024

ITEM_BLOCK = 2048
SUBLANES = 8


def _make_sc_gather(num_users):
    info = plsc.get_sparse_core_info()
    nc = info.num_cores
    b_per_c = BATCH // nc
    mesh = plsc.ScalarSubcoreMesh(axis_name="c", num_cores=nc)

    @functools.partial(
        pl.kernel,
        mesh=mesh,
        out_type=jax.ShapeDtypeStruct((BATCH, REQ_VEC), jnp.float32),
        scratch_types=[
            pltpu.SMEM((b_per_c,), jnp.int32),
            pltpu.SemaphoreType.DMA,
        ],
    )
    def gather_kernel(table_hbm, idx_hbm, out_hbm, idx_s, sem):
        base = lax.axis_index("c") * b_per_c
        pltpu.sync_copy(idx_hbm.at[pl.ds(base, b_per_c)], idx_s)

        def issue(i, _):
            pltpu.make_async_copy(
                table_hbm.at[idx_s[i]], out_hbm.at[base + i], sem
            ).start()
            return ()

        def drain(i, _):
            pltpu.make_async_copy(
                table_hbm.at[idx_s[i]], out_hbm.at[base + i], sem
            ).wait()
            return ()

        lax.fori_loop(0, b_per_c, issue, ())
        lax.fori_loop(0, b_per_c, drain, ())

    return gather_kernel


def _score_body(ug_ref, item_ref, fs_ref, out_ref, fu_ref):
    @pl.when(pl.program_id(0) == 0)
    def _():
        fu_ref[...] = jnp.dot(
            ug_ref[...], fs_ref[...], preferred_element_type=jnp.float32
        )

    fi = jnp.dot(item_ref[...], fs_ref[...], preferred_element_type=jnp.float32)
    logits = lax.dot_general(
        fu_ref[...], fi, (((1,), (1,)), ((), ())),
        preferred_element_type=jnp.float32,
    )
    out_ref[...] = jax.nn.sigmoid(logits)


def _tc_score(ug, item_vector, FS, interpret=False):
    n_items = item_vector.shape[0]
    grid = (pl.cdiv(n_items, ITEM_BLOCK),)
    return pl.pallas_call(
        _score_body,
        grid=grid,
        in_specs=[
            pl.BlockSpec((BATCH, REQ_VEC), lambda i: (0, 0)),
            pl.BlockSpec((ITEM_BLOCK, REQ_VEC), lambda i: (i, 0)),
            pl.BlockSpec((REQ_VEC, LATENT), lambda i: (0, 0)),
        ],
        out_specs=pl.BlockSpec((BATCH, ITEM_BLOCK), lambda i: (0, i)),
        out_shape=jax.ShapeDtypeStruct((BATCH, n_items), jnp.float32),
        scratch_shapes=[pltpu.VMEM((BATCH, LATENT), jnp.float32)],
        interpret=interpret,
    )(ug, item_vector, FS)


@jax.jit
def kernel(users, user_vector, item_vector, FS):
    users = users.astype(jnp.int32)
    gather = _make_sc_gather(user_vector.shape[0])
    ug = gather(user_vector, users)
    return _tc_score(ug, item_vector, FS)
